# baseline (speedup 1.0000x reference)
import functools
import jax
import jax.numpy as jnp
from jax.experimental import pallas as pl
from jax.experimental.pallas import tpu as pltpu

_B, _N, _T = 2, 10000, 10000
_H, _C = 4, 64
_NBLK = 512
_F32 = jnp.float32


def _ln_in(x, g, b, eps=1e-5):
    m = x.mean(-1, keepdims=True)
    v = ((x - m) ** 2).mean(-1, keepdims=True)
    return (x - m) / jnp.sqrt(v + eps) * g + b


def _silu(x):
    return x * jax.nn.sigmoid(x)


def _node_dense_kernel(descs_ref, uvd_ref,
                       pw1_ref, pb1_ref, pg1_ref, pbb1_ref,
                       pw2_ref, pb2_ref, pg2_ref, pbb2_ref,
                       gld_ref, glp_ref, glb_ref,
                       grd_ref, grp_ref, grb_ref,
                       rpd_ref, rpp_ref, rpb_ref,
                       xl_ref, xr_ref, rpx_ref):
    x = descs_ref[...]
    uvd = uvd_ref[...]
    h = jnp.dot(uvd, pw1_ref[...], preferred_element_type=_F32, precision=jax.lax.Precision.HIGHEST) + pb1_ref[...]
    h = _silu(_ln_in(h, pg1_ref[...], pbb1_ref[...]))
    pf = jnp.dot(h, pw2_ref[...], preferred_element_type=_F32, precision=jax.lax.Precision.HIGHEST) + pb2_ref[...]
    pf = _ln_in(pf, pg2_ref[...], pbb2_ref[...])
    xl_ref[...] = (jnp.dot(x, gld_ref[...], preferred_element_type=_F32, precision=jax.lax.Precision.HIGHEST)
                   + jnp.dot(pf, glp_ref[...], preferred_element_type=_F32, precision=jax.lax.Precision.HIGHEST)
                   + glb_ref[...])
    xr_ref[...] = (jnp.dot(x, grd_ref[...], preferred_element_type=_F32, precision=jax.lax.Precision.HIGHEST)
                   + jnp.dot(pf, grp_ref[...], preferred_element_type=_F32, precision=jax.lax.Precision.HIGHEST)
                   + grb_ref[...])
    rpx_ref[...] = (jnp.dot(x, rpd_ref[...], preferred_element_type=_F32, precision=jax.lax.Precision.HIGHEST)
                    + jnp.dot(pf, rpp_ref[...], preferred_element_type=_F32, precision=jax.lax.Precision.HIGHEST)
                    + rpb_ref[...])


def _node_dense(descs2, uvd, p):
    bn = descs2.shape[0]
    grid = (pl.cdiv(bn, _NBLK),)

    def full(a):
        return pl.BlockSpec(a.shape, lambda i: tuple(0 for _ in a.shape))

    ins = [descs2, uvd,
           p['pe_w1'].T, p['pe_b1'][None], p['pe_ln1_g'][None], p['pe_ln1_b'][None],
           p['pe_w2'].T, p['pe_b2'][None], p['pe_ln2_g'][None], p['pe_ln2_b'][None],
           p['gl_w'][:, :256].T, p['gl_w'][:, 256:].T, p['gl_b'][None],
           p['gr_w'][:, :256].T, p['gr_w'][:, 256:].T, p['gr_b'][None],
           p['rp_w'][:, :256].T, p['rp_w'][:, 256:].T, p['rp_b'][None]]
    in_specs = [pl.BlockSpec((_NBLK, 256), lambda i: (i, 0)),
                pl.BlockSpec((_NBLK, 3), lambda i: (i, 0))] + [full(a) for a in ins[2:]]
    out_shape = [jax.ShapeDtypeStruct((bn, 256), _F32)] * 3
    out_specs = [pl.BlockSpec((_NBLK, 256), lambda i: (i, 0))] * 3
    return pl.pallas_call(_node_dense_kernel, grid=grid, in_specs=in_specs,
                          out_specs=out_specs, out_shape=out_shape)(*ins)


def _post_kernel(agg_ref, rpx_ref, gb_ref, ng_ref, nb_ref, prw_ref, prb_ref,
                 nf_ref):
    y = _ln_in(agg_ref[...] + gb_ref[...], ng_ref[...], nb_ref[...])
    z = _silu(y + rpx_ref[...])
    nf_ref[...] = jnp.dot(z, prw_ref[...], preferred_element_type=_F32, precision=jax.lax.Precision.HIGHEST) + prb_ref[...]


def _post_dense(agg, rpx, p):
    bn = agg.shape[0]
    grid = (pl.cdiv(bn, _NBLK),)

    def full(a):
        return pl.BlockSpec(a.shape, lambda i: tuple(0 for _ in a.shape))

    ins = [agg, rpx, p['g_bias'][None], p['n_g'][None], p['n_b'][None],
           p['pr_w'].T, p['pr_b'][None]]
    in_specs = [pl.BlockSpec((_NBLK, 256), lambda i: (i, 0))] * 2 + [full(a) for a in ins[2:]]
    return pl.pallas_call(_post_kernel, grid=grid, in_specs=in_specs,
                          out_specs=pl.BlockSpec((_NBLK, 256), lambda i: (i, 0)),
                          out_shape=jax.ShapeDtypeStruct((bn, 256), _F32))(*ins)


def _tri_kernel(f_ref, w1_ref, b1_ref, w2_ref, b2_ref,
                wh_ref, whb_ref, w_ref):
    f = f_ref[...]
    h = _silu(jnp.dot(f, w1_ref[...], preferred_element_type=_F32, precision=jax.lax.Precision.HIGHEST) + b1_ref[...])
    h = _silu(jnp.dot(h, w2_ref[...], preferred_element_type=_F32, precision=jax.lax.Precision.HIGHEST) + b2_ref[...])
    w_ref[...] = jax.nn.sigmoid(
        jnp.dot(h, wh_ref[...], preferred_element_type=_F32, precision=jax.lax.Precision.HIGHEST) + whb_ref[...])


def _tri_mlp(f, p):
    bt = f.shape[0]
    grid = (pl.cdiv(bt, _NBLK),)

    def full(a):
        return pl.BlockSpec(a.shape, lambda i: tuple(0 for _ in a.shape))

    ins = [f, p['th_w1'].T, p['th_b1'][None], p['th_w2'].T,
           p['th_b2'][None], p['wh_w'].T, p['wh_b'][None]]
    in_specs = [pl.BlockSpec((_NBLK, 768), lambda i: (i, 0))] + [full(a) for a in ins[1:]]
    return pl.pallas_call(_tri_kernel, grid=grid, in_specs=in_specs,
                          out_specs=pl.BlockSpec((_NBLK, 1), lambda i: (i, 0)),
                          out_shape=jax.ShapeDtypeStruct((bt, 1), _F32))(*ins)


def kernel(descs, kpts, pts_3d, tri_indices, kpts_tp1, intrinsics, params):
    p = params
    bn = _B * _N

    srcs, dsts = [], []
    for b in range(_B):
        t = tri_indices[b]
        i, j, k = t[:, 0], t[:, 1], t[:, 2]
        srcs.append(jnp.concatenate([i, j, j, k, k, i]) + b * _N)
        dsts.append(jnp.concatenate([j, i, k, j, i, k]) + b * _N)
    edge_index = jnp.stack([jnp.concatenate(srcs), jnp.concatenate(dsts)])
    s, d = edge_index[0], edge_index[1]

    kp = kpts.reshape(-1, 2)
    p3 = pts_3d.reshape(-1, 3)
    norm_uv = kp / jnp.array([1216.0, 352.0], _F32)
    depth = jnp.clip(p3[:, 2:3], 0.1, 100.0)
    uvd = jnp.concatenate([norm_uv, depth], -1)

    xl, xr, rpx = _node_dense(descs.reshape(-1, 256), uvd, p)

    rel_uv = norm_uv[d] - norm_uv[s]
    dist = jnp.linalg.norm(rel_uv, axis=-1, keepdims=True)
    edge_attr = jnp.concatenate([rel_uv, dist], -1)

    xl3 = xl.reshape(-1, _H, _C)
    xr3 = xr.reshape(-1, _H, _C)
    e3 = (edge_attr @ p['ge_w'].T).reshape(-1, _H, _C)
    m = jax.nn.leaky_relu(xl3[s] + xr3[d] + e3, 0.2)
    logits = (m * p['att']).sum(-1)
    lmax = jax.ops.segment_max(logits, d, num_segments=bn)
    lmax = jnp.where(jnp.isfinite(lmax), lmax, 0.0)
    ex = jnp.exp(logits - lmax[d])
    den = jax.ops.segment_sum(ex, d, num_segments=bn)
    alpha = ex / (den[d] + 1e-16)
    agg = jax.ops.segment_sum(xl3[s] * alpha[..., None], d,
                              num_segments=bn).reshape(-1, _H * _C)

    node_feat = _post_dense(agg, rpx, p).reshape(_B, _N, 256)

    tri = tri_indices
    fparts = []
    for v in range(3):
        idx = jnp.broadcast_to(tri[:, :, v:v + 1], (_B, _T, 256))
        fparts.append(jnp.take_along_axis(node_feat, idx, axis=1))
    f = jnp.concatenate(fparts, -1).reshape(_B * _T, 768)

    w_j = _tri_mlp(f, p)

    nkey = jax.random.key(12345)
    Rs_l, cw_l, vc_l = [], [], []
    for b in range(_B):
        tris = tri_indices[b]
        fx, fy, cx, cy = intrinsics[b]
        ux = (kpts_tp1[b, :, 0] - cx) / (fx + 1e-8)
        uy = (kpts_tp1[b, :, 1] - cy) / (fy + 1e-8)
        pn = jnp.stack([ux, uy, jnp.ones_like(ux)], -1)
        q = pn[tris]
        pw = pts_3d[b][tris]
        K_base = jnp.einsum('tva,tvb->tab', q, pw)
        K_j = K_base + jax.random.normal(jax.random.fold_in(nkey, b),
                                         K_base.shape, _F32) * 1e-5
        U, S, Vt = jnp.linalg.svd(K_j, full_matrices=False)
        det = jnp.linalg.det(jnp.einsum('tij,tjk->tik', U, Vt))
        D = (jnp.zeros_like(K_j).at[:, 0, 0].set(1.0)
             .at[:, 1, 1].set(1.0).at[:, 2, 2].set(det))
        R_j = jnp.einsum('tij,tjk,tkl->til', U, D, Vt)
        r13 = R_j[:, 0, 2]
        r33 = R_j[:, 2, 2] + 1e-8
        xv_j = fx * (r13 / r33) + cx
        wv = w_j[b * _T:(b + 1) * _T, 0]
        mu = (wv * xv_j).sum() / (wv.sum() + 1e-8)
        kern = jnp.exp(-((xv_j - mu) ** 2) / 8.0)
        xv_star = (wv * kern * xv_j).sum() / ((wv * kern).sum() + 1e-8)
        s_j = jnp.exp(-((xv_j - xv_star) ** 2) / 8.0)[:, None]
        cw_b = w_j[b * _T:(b + 1) * _T] * s_j
        Kw = jnp.einsum('t,tab->ab', cw_b[:, 0], K_base)
        Uw, Sw, Vtw = jnp.linalg.svd(Kw, full_matrices=False)
        dw = jnp.linalg.det(Uw @ Vtw)
        Dw = jnp.diag(jnp.stack([1.0 + 0.0 * dw, 1.0 + 0.0 * dw, dw]))
        Rs_l.append(Uw @ Dw @ Vtw)
        cw_l.append(cw_b)
        vc_l.append(jnp.tanh(jnp.zeros((_N,), _F32)
                             .at[tris.reshape(-1)].add(jnp.repeat(s_j[:, 0], 3)))[:, None])

    return (jnp.stack(Rs_l), jnp.stack(cw_l), jnp.stack(vc_l),
            edge_index, edge_attr)
